# baseline (speedup 1.0000x reference)
import jax
import jax.numpy as jnp
import numpy as np
from jax.experimental import pallas as pl

_N = 90
_C = 96
_HD = 24
_E = 5
_NBLK = 6
_BB = 32


def _adj_bias_const():
    H, W = 10, 9
    deltas = {
        'adjacent': [(-1, 0), (1, 0), (0, -1), (0, 1),
                     (-1, -1), (-1, 1), (1, -1), (1, 1)],
        'knight': [(-2, -1), (-2, 1), (-1, -2), (-1, 2),
                   (1, -2), (1, 2), (2, -1), (2, 1)],
        'elephant': [(-2, -2), (-2, 2), (2, -2), (2, 2)],
    }
    bias = np.full((_E, _N, _N), -1e9, dtype=np.float32)
    for ei, et in enumerate(['adjacent', 'row', 'col', 'knight', 'elephant']):
        for r in range(H):
            for c in range(W):
                i = r * W + c
                if et == 'row':
                    for nc in range(W):
                        if abs(nc - c) > 1:
                            bias[ei, i, r * W + nc] = 0.0
                elif et == 'col':
                    for nr in range(H):
                        if abs(nr - r) > 1:
                            bias[ei, i, nr * W + c] = 0.0
                else:
                    for dr, dc in deltas[et]:
                        nr, nc = r + dr, c + dc
                        if 0 <= nr < H and 0 <= nc < W:
                            bias[ei, i, nr * W + nc] = 0.0
    return bias


_ADJ_BIAS = _adj_bias_const()


def _trunk_kernel(xr, adjb, ipw, ipb, attw, attb, ew, eb,
                  w1a, w1b, w1c, csum, m1bf, m2w, m2b,
                  s1w, s1b, s2w, s2b,
                  p1w, p1b, v1w, v1b, v2w, v2b, t1w, t1b, t2w, t2b,
                  p_out, v_out, m_out):
    b = _BB
    hf = jnp.maximum(xr[...].reshape(b * _N, -1) @ ipw[...] + ipb[...], 0.0)
    for blk in range(_NBLK):
        h3 = hf.reshape(b, _N, _C)
        gb = jnp.mean(h3, axis=1)
        agg = jnp.zeros((b * _N, _C), jnp.float32)
        for e in range(_E):
            hx = (hf @ attw[blk, e] + attb[blk, e]).reshape(b, _N, _C + 1)
            logits = jax.lax.dot_general(
                hx[:, :, :_C], h3, (((2,), (2,)), ((0,), (0,))))
            logits = logits + hx[:, :, _C:] + adjb[e][None]
            logits = logits - jnp.max(logits, axis=-1, keepdims=True)
            ex = jnp.exp(logits)
            w = ex * jax.lax.reciprocal(jnp.sum(ex, axis=-1, keepdims=True))
            nw = jax.lax.dot_general(
                w, h3, (((2,), (1,)), ((0,), (0,))))
            agg = agg + (nw.reshape(b * _N, _C) @ ew[blk, e] + eb[blk, e])
        gs1 = jnp.sum(gb, axis=-1, keepdims=True)
        gs2 = jnp.sum(gb * gb, axis=-1, keepdims=True)
        rs1 = (jnp.sum(hf, axis=-1, keepdims=True)
               + jnp.sum(agg, axis=-1, keepdims=True)
               + jnp.broadcast_to(gs1[:, None, :], (b, _N, 1)).reshape(b * _N, 1))
        rs2 = (jnp.sum(hf * hf, axis=-1, keepdims=True)
               + jnp.sum(agg * agg, axis=-1, keepdims=True)
               + jnp.broadcast_to(gs2[:, None, :], (b, _N, 1)).reshape(b * _N, 1))
        mu = rs1 * (1.0 / (3 * _C))
        var = rs2 * (1.0 / (3 * _C)) - mu * mu
        inv = jax.lax.rsqrt(var + 1e-6)
        gz = gb @ w1c[blk]
        z = (hf @ w1a[blk] + agg @ w1b[blk]
             + jnp.broadcast_to(gz[:, None, :], (b, _N, _C)).reshape(b * _N, _C))
        t = jnp.maximum((z - mu * csum[blk]) * inv + m1bf[blk], 0.0)
        t = t @ m2w[blk] + m2b[blk]
        out = hf + t
        o3 = out.reshape(b, _N, _C)
        se = jnp.mean(o3, axis=1)
        se = jnp.maximum(se @ s1w[blk] + s1b[blk], 0.0)
        se = jax.nn.sigmoid(se @ s2w[blk] + s2b[blk])
        hf = (o3 * se[:, None, :]).reshape(b * _N, _C)
    p_out[...] = jnp.maximum(hf @ p1w[...] + p1b[...], 0.0).reshape(b, _N, 32)
    hm = jnp.mean(hf.reshape(b, _N, _C), axis=1)
    v = jnp.maximum(hm @ v1w[...] + v1b[...], 0.0)
    v_out[...] = jnp.tanh(v @ v2w[...] + v2b[...])
    m = jnp.maximum(hm @ t1w[...] + t1b[...], 0.0)
    m_out[...] = jnp.tanh(m @ t2w[...] + t2b[...])


def _policy2_kernel(pf, w, bias, out):
    out[...] = pf[...] @ w[...] + bias[...]


def kernel(x, params, edge_indices, edge_masks):
    B = x.shape[0]
    xr = jnp.transpose(x, (0, 2, 3, 1)).reshape(B, _N, -1)
    adjb = jnp.asarray(_ADJ_BIAS)

    blks = params['blocks']
    st = lambda f: jnp.stack([f(bp) for bp in blks])
    ste = lambda key, leaf: st(
        lambda bp: jnp.stack([bp[key][e][leaf] for e in range(_E)]))
    qw, qb = ste('attn_q', 'w'), ste('attn_q', 'b')
    kw, kb = ste('attn_k', 'w'), ste('attn_k', 'b')
    ew, eb = ste('edge_proj', 'w'), ste('edge_proj', 'b')
    lns, lnb = st(lambda bp: bp['ln']['scale']), st(lambda bp: bp['ln']['bias'])
    m1w, m1b = st(lambda bp: bp['mlp1']['w']), st(lambda bp: bp['mlp1']['b'])
    m2w, m2b = st(lambda bp: bp['mlp2']['w']), st(lambda bp: bp['mlp2']['b'])
    s1w, s1b = st(lambda bp: bp['se1']['w']), st(lambda bp: bp['se1']['b'])
    s2w, s2b = st(lambda bp: bp['se2']['w']), st(lambda bp: bp['se2']['b'])

    scale = _HD ** -0.5
    mm = jnp.einsum('bewh,bevh->bewv', qw, kw) * scale
    uu = jnp.einsum('bewh,beh->bew', qw, kb) * scale
    rr = jnp.einsum('bewh,beh->bew', kw, qb) * scale
    cc = jnp.einsum('beh,beh->be', qb, kb) * scale
    attw = jnp.concatenate([mm, uu[..., None]], axis=-1)
    attb = jnp.concatenate([rr, cc[..., None]], axis=-1)

    w1s = lns[..., None] * m1w
    m1bf = jnp.einsum('bk,bkc->bc', lnb, m1w) + m1b
    w1a, w1b_, w1c = w1s[:, :_C], w1s[:, _C:2 * _C], w1s[:, 2 * _C:]
    csum = jnp.sum(w1s, axis=1)

    consts = [adjb,
              params['in_proj']['w'], params['in_proj']['b'],
              attw, attb, ew, eb,
              w1a, w1b_, w1c, csum, m1bf, m2w, m2b,
              s1w, s1b, s2w, s2b,
              params['policy1']['w'], params['policy1']['b'],
              params['value1']['w'], params['value1']['b'],
              params['value2']['w'], params['value2']['b'],
              params['mat1']['w'], params['mat1']['b'],
              params['mat2']['w'], params['mat2']['b']]

    def _full(a):
        nd = a.ndim
        return pl.BlockSpec(a.shape, lambda i, _nd=nd: (0,) * _nd)

    p32, vq, mat = pl.pallas_call(
        _trunk_kernel,
        grid=(B // _BB,),
        in_specs=[pl.BlockSpec((_BB, _N, xr.shape[-1]), lambda i: (i, 0, 0))]
                 + [_full(a) for a in consts],
        out_specs=[pl.BlockSpec((_BB, _N, 32), lambda i: (i, 0, 0)),
                   pl.BlockSpec((_BB, 64), lambda i: (i, 0)),
                   pl.BlockSpec((_BB, 1), lambda i: (i, 0))],
        out_shape=[jax.ShapeDtypeStruct((B, _N, 32), jnp.float32),
                   jax.ShapeDtypeStruct((B, 64), jnp.float32),
                   jax.ShapeDtypeStruct((B, 1), jnp.float32)],
    )(xr, *consts)

    pfeat = p32.reshape(B, _N * 32)
    p2w = params['policy2']['w']
    p2b = params['policy2']['b'].reshape(1, -1)
    nact = p2w.shape[1]
    nblocks = (nact + 127) // 128
    pol = pl.pallas_call(
        _policy2_kernel,
        grid=(nblocks,),
        in_specs=[pl.BlockSpec((B, _N * 32), lambda j: (0, 0)),
                  pl.BlockSpec((_N * 32, 128), lambda j: (0, j)),
                  pl.BlockSpec((1, 128), lambda j: (0, j))],
        out_specs=pl.BlockSpec((B, 128), lambda j: (0, j)),
        out_shape=jax.ShapeDtypeStruct((B, nact), jnp.float32),
    )(pfeat, p2w, p2b)

    return (pol.astype(jnp.float32), vq.astype(jnp.float32),
            mat[:, 0].astype(jnp.float32))


# speedup vs baseline: 1.2638x; 1.2638x over previous
import jax
import jax.numpy as jnp
import numpy as np
from jax.experimental import pallas as pl

_N = 90
_C = 96
_HD = 24
_E = 5
_NBLK = 6
_BB = 32


def _adj_bias_const():
    H, W = 10, 9
    deltas = {
        'adjacent': [(-1, 0), (1, 0), (0, -1), (0, 1),
                     (-1, -1), (-1, 1), (1, -1), (1, 1)],
        'knight': [(-2, -1), (-2, 1), (-1, -2), (-1, 2),
                   (1, -2), (1, 2), (2, -1), (2, 1)],
        'elephant': [(-2, -2), (-2, 2), (2, -2), (2, 2)],
    }
    bias = np.full((_E, _N, _N), -1e9, dtype=np.float32)
    for ei, et in enumerate(['adjacent', 'row', 'col', 'knight', 'elephant']):
        for r in range(H):
            for c in range(W):
                i = r * W + c
                if et == 'row':
                    for nc in range(W):
                        if abs(nc - c) > 1:
                            bias[ei, i, r * W + nc] = 0.0
                elif et == 'col':
                    for nr in range(H):
                        if abs(nr - r) > 1:
                            bias[ei, i, nr * W + c] = 0.0
                else:
                    for dr, dc in deltas[et]:
                        nr, nc = r + dr, c + dc
                        if 0 <= nr < H and 0 <= nc < W:
                            bias[ei, i, nr * W + nc] = 0.0
    return bias


_ADJ_BIAS = _adj_bias_const()


def _trunk_kernel(xr, adjb, ipw, ipb, attw, attb, ew, eb,
                  w1a, w1b, w1c, csum, m1bf, m2w, m2b,
                  s1w, s1b, s2w, s2b,
                  p1w, p1b, v1w, v1b, v2w, v2b, t1w, t1b, t2w, t2b,
                  p_out, v_out, m_out):
    b = _BB
    hf = jnp.maximum(xr[...].reshape(b * _N, -1) @ ipw[...] + ipb[...], 0.0)
    for blk in range(_NBLK):
        h3 = hf.reshape(b, _N, _C)
        gb = jnp.mean(h3, axis=1)
        hx_all = jnp.concatenate(
            [(hf @ attw[blk, e] + attb[blk, e]).reshape(b, _N, _C + 1)
             for e in range(_E)], axis=1)
        logits = jax.lax.dot_general(
            hx_all[:, :, :_C], h3, (((2,), (2,)), ((0,), (0,))))
        logits = logits + hx_all[:, :, _C:] + adjb[...][None]
        logits = logits - jnp.max(logits, axis=-1, keepdims=True)
        ex = jnp.exp(logits)
        w = ex * jax.lax.reciprocal(jnp.sum(ex, axis=-1, keepdims=True))
        nw_all = jax.lax.dot_general(
            w, h3, (((2,), (1,)), ((0,), (0,))))
        agg = jnp.broadcast_to(eb[blk], (b * _N, _C))
        for e in range(_E):
            nw = nw_all[:, e * _N:(e + 1) * _N, :].reshape(b * _N, _C)
            agg = agg + nw @ ew[blk, e]
        gs1 = jnp.sum(gb, axis=-1, keepdims=True)
        gs2 = jnp.sum(gb * gb, axis=-1, keepdims=True)
        rs1 = (jnp.sum(hf, axis=-1, keepdims=True)
               + jnp.sum(agg, axis=-1, keepdims=True)
               + jnp.broadcast_to(gs1[:, None, :], (b, _N, 1)).reshape(b * _N, 1))
        rs2 = (jnp.sum(hf * hf, axis=-1, keepdims=True)
               + jnp.sum(agg * agg, axis=-1, keepdims=True)
               + jnp.broadcast_to(gs2[:, None, :], (b, _N, 1)).reshape(b * _N, 1))
        mu = rs1 * (1.0 / (3 * _C))
        var = rs2 * (1.0 / (3 * _C)) - mu * mu
        inv = jax.lax.rsqrt(var + 1e-6)
        gz = gb @ w1c[blk]
        z = (hf @ w1a[blk] + agg @ w1b[blk]
             + jnp.broadcast_to(gz[:, None, :], (b, _N, _C)).reshape(b * _N, _C))
        t = jnp.maximum((z - mu * csum[blk]) * inv + m1bf[blk], 0.0)
        t = t @ m2w[blk] + m2b[blk]
        out = hf + t
        o3 = out.reshape(b, _N, _C)
        se = jnp.mean(o3, axis=1)
        se = jnp.maximum(se @ s1w[blk] + s1b[blk], 0.0)
        se = jax.nn.sigmoid(se @ s2w[blk] + s2b[blk])
        hf = (o3 * se[:, None, :]).reshape(b * _N, _C)
    p_out[...] = jnp.maximum(hf @ p1w[...] + p1b[...], 0.0).reshape(b, _N, 32)
    hm = jnp.mean(hf.reshape(b, _N, _C), axis=1)
    v = jnp.maximum(hm @ v1w[...] + v1b[...], 0.0)
    v_out[...] = jnp.tanh(v @ v2w[...] + v2b[...])
    m = jnp.maximum(hm @ t1w[...] + t1b[...], 0.0)
    m_out[...] = jnp.tanh(m @ t2w[...] + t2b[...])


def _policy2_kernel(pf, w, bias, out):
    out[...] = pf[...] @ w[...] + bias[...]


def kernel(x, params, edge_indices, edge_masks):
    B = x.shape[0]
    xr = jnp.transpose(x, (0, 2, 3, 1)).reshape(B, _N, -1)
    adjb = jnp.asarray(_ADJ_BIAS.reshape(_E * _N, _N))

    blks = params['blocks']
    st = lambda f: jnp.stack([f(bp) for bp in blks])
    ste = lambda key, leaf: st(
        lambda bp: jnp.stack([bp[key][e][leaf] for e in range(_E)]))
    qw, qb = ste('attn_q', 'w'), ste('attn_q', 'b')
    kw, kb = ste('attn_k', 'w'), ste('attn_k', 'b')
    ew, eb = ste('edge_proj', 'w'), ste('edge_proj', 'b')
    eb = jnp.sum(eb, axis=1)
    lns, lnb = st(lambda bp: bp['ln']['scale']), st(lambda bp: bp['ln']['bias'])
    m1w, m1b = st(lambda bp: bp['mlp1']['w']), st(lambda bp: bp['mlp1']['b'])
    m2w, m2b = st(lambda bp: bp['mlp2']['w']), st(lambda bp: bp['mlp2']['b'])
    s1w, s1b = st(lambda bp: bp['se1']['w']), st(lambda bp: bp['se1']['b'])
    s2w, s2b = st(lambda bp: bp['se2']['w']), st(lambda bp: bp['se2']['b'])

    scale = _HD ** -0.5
    mm = jnp.einsum('bewh,bevh->bewv', qw, kw) * scale
    uu = jnp.einsum('bewh,beh->bew', qw, kb) * scale
    rr = jnp.einsum('bewh,beh->bew', kw, qb) * scale
    cc = jnp.einsum('beh,beh->be', qb, kb) * scale
    attw = jnp.concatenate([mm, uu[..., None]], axis=-1)
    attb = jnp.concatenate([rr, cc[..., None]], axis=-1)

    w1s = lns[..., None] * m1w
    m1bf = jnp.einsum('bk,bkc->bc', lnb, m1w) + m1b
    w1a, w1b_, w1c = w1s[:, :_C], w1s[:, _C:2 * _C], w1s[:, 2 * _C:]
    csum = jnp.sum(w1s, axis=1)

    consts = [adjb,
              params['in_proj']['w'], params['in_proj']['b'],
              attw, attb, ew, eb,
              w1a, w1b_, w1c, csum, m1bf, m2w, m2b,
              s1w, s1b, s2w, s2b,
              params['policy1']['w'], params['policy1']['b'],
              params['value1']['w'], params['value1']['b'],
              params['value2']['w'], params['value2']['b'],
              params['mat1']['w'], params['mat1']['b'],
              params['mat2']['w'], params['mat2']['b']]

    def _full(a):
        nd = a.ndim
        return pl.BlockSpec(a.shape, lambda i, _nd=nd: (0,) * _nd)

    p32, vq, mat = pl.pallas_call(
        _trunk_kernel,
        grid=(B // _BB,),
        in_specs=[pl.BlockSpec((_BB, _N, xr.shape[-1]), lambda i: (i, 0, 0))]
                 + [_full(a) for a in consts],
        out_specs=[pl.BlockSpec((_BB, _N, 32), lambda i: (i, 0, 0)),
                   pl.BlockSpec((_BB, 64), lambda i: (i, 0)),
                   pl.BlockSpec((_BB, 1), lambda i: (i, 0))],
        out_shape=[jax.ShapeDtypeStruct((B, _N, 32), jnp.float32),
                   jax.ShapeDtypeStruct((B, 64), jnp.float32),
                   jax.ShapeDtypeStruct((B, 1), jnp.float32)],
    )(xr, *consts)

    pfeat = p32.reshape(B, _N * 32)
    p2w = params['policy2']['w']
    p2b = params['policy2']['b'].reshape(1, -1)
    nact = p2w.shape[1]
    nblocks = (nact + 127) // 128
    pol = pl.pallas_call(
        _policy2_kernel,
        grid=(nblocks,),
        in_specs=[pl.BlockSpec((B, _N * 32), lambda j: (0, 0)),
                  pl.BlockSpec((_N * 32, 128), lambda j: (0, j)),
                  pl.BlockSpec((1, 128), lambda j: (0, j))],
        out_specs=pl.BlockSpec((B, 128), lambda j: (0, j)),
        out_shape=jax.ShapeDtypeStruct((B, nact), jnp.float32),
    )(pfeat, p2w, p2b)

    return (pol.astype(jnp.float32), vq.astype(jnp.float32),
            mat[:, 0].astype(jnp.float32))
